# Initial kernel scaffold; baseline (speedup 1.0000x reference)
import jax
import jax.numpy as jnp
from jax.experimental import pallas as pl


def kernel(x, edge_index, edge_weight, W1, b1, W2, b2, W3, b3, Wfc, bfc):
    raise NotImplementedError("write your pallas kernel here")


# speedup vs baseline: 6.1958x; 6.1958x over previous
import dataclasses
import functools

import jax
import jax.numpy as jnp
from jax import lax
from jax.experimental import pallas as pl
from jax.experimental.pallas import tpu as pltpu
from jax.experimental.pallas import tpu_sc as plsc

N = 10000
E = 320000
FIN = 128
H = 256
OUT = 2

NP = 10240
CH = 128
EC = E // CH
NSUB = 16
NCORE = 2
BN = 1024
NB = NP // BN

_mesh = plsc.VectorSubcoreMesh(core_axis_name="c", subcore_axis_name="s")

_sc_params = pltpu.CompilerParams()
if "needs_layout_passes" in pltpu.CompilerParams.__dataclass_fields__:
    _sc_params = dataclasses.replace(_sc_params, needs_layout_passes=False)


def _zero_vmem2d(buf, rows):
    z = jnp.zeros((16,), jnp.float32)

    @pl.loop(0, rows)
    def _(e):
        for j in range(8):
            buf[e, pl.ds(16 * j, 16)] = z


def _rsqrt16(d):
    i = plsc.bitcast(d, jnp.int32)
    i = 0x5F3759DF - lax.shift_right_arithmetic(i, 1)
    y = plsc.bitcast(i, jnp.float32)
    for _ in range(3):
        y = y * (1.5 - 0.5 * d * y * y)
    return y


def _prep_body(row_hbm, col_hbm, ew_hbm, norm_hbm, selfw_hbm,
               deg_sp, zbuf, colbuf, ewbuf, rowbuf, nbuf, swbuf,
               degbuf, dinvbuf):
    s = lax.axis_index("s")
    c = lax.axis_index("c")

    @pl.loop(0, 40)
    def _(i):
        zbuf[pl.ds(16 * i, 16)] = jnp.zeros((16,), jnp.float32)

    pltpu.sync_copy(zbuf, deg_sp.at[pl.ds(s * 640, 640)])
    plsc.subcore_barrier()

    @pl.loop(0, 157)
    def _(i):
        chunk = s + 16 * i

        @pl.when(chunk < EC)
        def _():
            pltpu.sync_copy(col_hbm.at[pl.ds(chunk, 1)], colbuf)
            pltpu.sync_copy(ew_hbm.at[pl.ds(chunk, 1)], ewbuf)
            pltpu.sync_copy(ewbuf.at[0], deg_sp.at[colbuf.at[0]], add=True)

    plsc.subcore_barrier()

    pltpu.sync_copy(deg_sp, degbuf)

    @pl.loop(0, NP // 16)
    def _(i):
        d = degbuf[pl.ds(16 * i, 16)] + 1.0
        dinvbuf[pl.ds(16 * i, 16)] = _rsqrt16(d)

    @pl.when(c == 0)
    def _():
        for m in range(5):
            r = s * 5 + m
            for j in range(8):
                v = dinvbuf[pl.ds(r * 128 + 16 * j, 16)]
                swbuf[0, pl.ds(16 * j, 16)] = v * v
            pltpu.sync_copy(swbuf, selfw_hbm.at[pl.ds(r, 1)])

    @pl.loop(0, 79)
    def _(i):
        t = s + 16 * i

        @pl.when(t < EC // 2)
        def _():
            chunk = c * (EC // 2) + t
            pltpu.sync_copy(row_hbm.at[pl.ds(chunk, 1)], rowbuf)
            pltpu.sync_copy(col_hbm.at[pl.ds(chunk, 1)], colbuf)
            pltpu.sync_copy(ew_hbm.at[pl.ds(chunk, 1)], ewbuf)
            for j in range(8):
                sl = pl.ds(16 * j, 16)
                dr = plsc.load_gather(dinvbuf, [rowbuf[0, sl]])
                dc = plsc.load_gather(dinvbuf, [colbuf[0, sl]])
                nbuf[0, sl] = dr * ewbuf[0, sl] * dc
            pltpu.sync_copy(nbuf, norm_hbm.at[pl.ds(chunk, 1)])


@jax.jit
def _prep(row2d, col2d, ew2d):
    return pl.kernel(
        _prep_body,
        out_type=(jax.ShapeDtypeStruct((EC, CH), jnp.float32),
                  jax.ShapeDtypeStruct((NP // CH, CH), jnp.float32)),
        mesh=_mesh,
        scratch_types=[
            pltpu.VMEM_SHARED((NP,), jnp.float32),
            pltpu.VMEM((640,), jnp.float32),
            pltpu.VMEM((1, CH), jnp.int32),
            pltpu.VMEM((1, CH), jnp.float32),
            pltpu.VMEM((1, CH), jnp.int32),
            pltpu.VMEM((1, CH), jnp.float32),
            pltpu.VMEM((1, CH), jnp.float32),
            pltpu.VMEM((NP,), jnp.float32),
            pltpu.VMEM((NP,), jnp.float32),
        ],
        compiler_params=_sc_params,
    )(row2d, col2d, ew2d)


def _layer_body(xw_hbm, row_hbm, col_hbm, norm_hbm, selfw_hbm, b_hbm,
                h_hbm, acc_sp, rowsbuf, xwbuf, idxbuf, colbuf, nbuf,
                swbuf, bbuf):
    s = lax.axis_index("s")
    c = lax.axis_index("c")

    _zero_vmem2d(rowsbuf, CH)
    for m in range(5):
        pltpu.sync_copy(rowsbuf, acc_sp.at[pl.ds(s * 640 + m * 128, 128)])
    pltpu.sync_copy(b_hbm.at[pl.ds(c, 1)], bbuf)
    plsc.subcore_barrier()

    @pl.loop(0, 157)
    def _(i):
        chunk = s + 16 * i

        @pl.when(chunk < EC)
        def _():
            pltpu.sync_copy(row_hbm.at[pl.ds(chunk, 1)], idxbuf)
            pltpu.sync_copy(norm_hbm.at[pl.ds(chunk, 1)], nbuf)
            pltpu.sync_copy(col_hbm.at[pl.ds(chunk, 1)], colbuf)
            pltpu.sync_copy(xw_hbm.at[c].at[idxbuf.at[0]], rowsbuf)

            @pl.loop(0, CH // 16)
            def _(g):
                nvec = nbuf[0, pl.ds(16 * g, 16)]
                for l in range(16):
                    ns = nvec[l]
                    e = 16 * g + l
                    for j in range(8):
                        sl = pl.ds(16 * j, 16)
                        rowsbuf[e, sl] = rowsbuf[e, sl] * ns

            pltpu.sync_copy(rowsbuf, acc_sp.at[colbuf.at[0]], add=True)

    plsc.subcore_barrier()

    for m in range(5):
        r0 = s * 640 + m * 128
        pltpu.sync_copy(acc_sp.at[pl.ds(r0, 128)], rowsbuf)
        pltpu.sync_copy(xw_hbm.at[c].at[pl.ds(r0, 128)], xwbuf)
        pltpu.sync_copy(selfw_hbm.at[pl.ds(s * 5 + m, 1)], swbuf)

        @pl.loop(0, CH // 16)
        def _(g):
            swv = swbuf[0, pl.ds(16 * g, 16)]
            for l in range(16):
                sw = swv[l]
                e = 16 * g + l
                for j in range(8):
                    sl = pl.ds(16 * j, 16)
                    v = rowsbuf[e, sl] + sw * xwbuf[e, sl] + bbuf[0, sl]
                    rowsbuf[e, sl] = jnp.maximum(v, 0.0)

        pltpu.sync_copy(rowsbuf, h_hbm.at[c].at[pl.ds(r0, 128)])


@jax.jit
def _layer(xw, row2d, col2d, norm2d, selfw, b2d):
    return pl.kernel(
        _layer_body,
        out_type=jax.ShapeDtypeStruct((NCORE, NP, 128), jnp.float32),
        mesh=_mesh,
        scratch_types=[
            pltpu.VMEM_SHARED((NP, 128), jnp.float32),
            pltpu.VMEM((CH, 128), jnp.float32),
            pltpu.VMEM((CH, 128), jnp.float32),
            pltpu.VMEM((1, CH), jnp.int32),
            pltpu.VMEM((1, CH), jnp.int32),
            pltpu.VMEM((1, CH), jnp.float32),
            pltpu.VMEM((1, CH), jnp.float32),
            pltpu.VMEM((1, 128), jnp.float32),
        ],
        compiler_params=_sc_params,
    )(xw, row2d, col2d, norm2d, selfw, b2d)


def _mm_split_body(a_ref, w_ref, o_ref):
    k = pl.program_id(2)

    @pl.when(k == 0)
    def _():
        o_ref[...] = jnp.zeros_like(o_ref)

    o_ref[0] += jnp.dot(a_ref[0], w_ref[0, 0],
                        preferred_element_type=jnp.float32)


@jax.jit
def _mm_split(parts, w4):
    kin = parts.shape[0]
    return pl.pallas_call(
        _mm_split_body,
        grid=(NB, NCORE, kin),
        in_specs=[
            pl.BlockSpec((1, BN, 128), lambda i, c, k: (k, i, 0)),
            pl.BlockSpec((1, 1, 128, 128), lambda i, c, k: (k, c, 0, 0)),
        ],
        out_specs=pl.BlockSpec((1, BN, 128), lambda i, c, k: (c, i, 0)),
        out_shape=jax.ShapeDtypeStruct((NCORE, NP, 128), jnp.float32),
    )(parts, w4)


def _mm_head_body(p_ref, w_ref, b_ref, o_ref):
    k = pl.program_id(1)

    @pl.when(k == 0)
    def _():
        o_ref[...] = jnp.broadcast_to(b_ref[...], o_ref.shape)

    o_ref[...] += jnp.dot(p_ref[0], w_ref[0],
                          preferred_element_type=jnp.float32)


@jax.jit
def _mm_head(parts, w, b):
    return pl.pallas_call(
        _mm_head_body,
        grid=(NB, 4),
        in_specs=[
            pl.BlockSpec((1, BN, 128), lambda i, k: (k, i, 0)),
            pl.BlockSpec((1, 128, 128), lambda i, k: (k, 0, 0)),
            pl.BlockSpec((1, 128), lambda i, k: (0, 0)),
        ],
        out_specs=pl.BlockSpec((BN, 128), lambda i, k: (i, 0)),
        out_shape=jax.ShapeDtypeStruct((NP, 128), jnp.float32),
    )(parts, w, b)


def kernel(x, edge_index, edge_weight, W1, b1, W2, b2, W3, b3, Wfc, bfc):
    row2d = edge_index[0].reshape(EC, CH)
    col2d = edge_index[1].reshape(EC, CH)
    ew2d = edge_weight.reshape(EC, CH)

    norm2d, selfw = _prep(row2d, col2d, ew2d)

    xp = jnp.zeros((1, NP, FIN), jnp.float32).at[0, :N, :].set(x)
    xw1 = _mm_split(xp, W1.reshape(1, 128, 2, 128).transpose(0, 2, 1, 3))
    h1 = _layer(xw1, row2d, col2d, norm2d, selfw, b1.reshape(2, 128))

    xw2 = _mm_split(h1, W2.reshape(2, 128, 2, 128).transpose(0, 2, 1, 3))
    h2 = _layer(xw2, row2d, col2d, norm2d, selfw, b2.reshape(2, 128))

    xw3 = _mm_split(h2, W3.reshape(2, 128, 2, 128).transpose(0, 2, 1, 3))
    h3 = _layer(xw3, row2d, col2d, norm2d, selfw, b3.reshape(2, 128))

    parts = jnp.concatenate([h2, h3], axis=0)
    wfc = jnp.zeros((4, 128, 128), jnp.float32).at[:, :, :OUT].set(
        Wfc.reshape(4, 128, OUT))
    bfc_pad = jnp.zeros((1, 128), jnp.float32).at[0, :OUT].set(bfc)
    out_pad = _mm_head(parts, wfc, bfc_pad)
    return out_pad[:N, :OUT]


# speedup vs baseline: 11.4526x; 1.8484x over previous
import dataclasses
import functools

import jax
import jax.numpy as jnp
from jax import lax
from jax.experimental import pallas as pl
from jax.experimental.pallas import tpu as pltpu
from jax.experimental.pallas import tpu_sc as plsc

N = 10000
E = 320000
FIN = 128
H = 256
OUT = 2

NP = 10240
CH = 128
EC = E // CH
G = 1
SCN = EC // G
NSUB = 16
NCORE = 2
BN = 1024
NB = NP // BN

_mesh = plsc.VectorSubcoreMesh(core_axis_name="c", subcore_axis_name="s")

_sc_params = pltpu.CompilerParams()
if "needs_layout_passes" in pltpu.CompilerParams.__dataclass_fields__:
    _sc_params = dataclasses.replace(_sc_params, needs_layout_passes=False)


def _zero_vmem2d(buf, rows):
    z = jnp.zeros((16,), jnp.float32)

    @pl.loop(0, rows)
    def _(e):
        for j in range(8):
            buf[e, pl.ds(16 * j, 16)] = z


def _rsqrt16(d):
    i = plsc.bitcast(d, jnp.int32)
    i = 0x5F3759DF - lax.shift_right_arithmetic(i, 1)
    y = plsc.bitcast(i, jnp.float32)
    for _ in range(3):
        y = y * (1.5 - 0.5 * d * y * y)
    return y


def _prep_body(row_hbm, col_hbm, ew_hbm, norm_hbm, selfw_hbm,
               deg_sp, zbuf, colbuf, ewbuf, rowbuf, nbuf, swbuf,
               degbuf, dinvbuf):
    s = lax.axis_index("s")
    c = lax.axis_index("c")

    @pl.loop(0, 40)
    def _(i):
        zbuf[pl.ds(16 * i, 16)] = jnp.zeros((16,), jnp.float32)

    pltpu.sync_copy(zbuf, deg_sp.at[pl.ds(s * 640, 640)])
    plsc.subcore_barrier()

    @pl.loop(0, 157)
    def _(i):
        chunk = s + 16 * i

        @pl.when(chunk < EC)
        def _():
            pltpu.sync_copy(col_hbm.at[pl.ds(chunk, 1)], colbuf)
            pltpu.sync_copy(ew_hbm.at[pl.ds(chunk, 1)], ewbuf)
            pltpu.sync_copy(ewbuf.at[0], deg_sp.at[colbuf.at[0]], add=True)

    plsc.subcore_barrier()

    pltpu.sync_copy(deg_sp, degbuf)

    @pl.loop(0, NP // 16)
    def _(i):
        d = degbuf[pl.ds(16 * i, 16)] + 1.0
        dinvbuf[pl.ds(16 * i, 16)] = _rsqrt16(d)

    @pl.when(c == 0)
    def _():
        for m in range(5):
            r = s * 5 + m
            for j in range(8):
                v = dinvbuf[pl.ds(r * 128 + 16 * j, 16)]
                swbuf[0, pl.ds(16 * j, 16)] = v * v
            pltpu.sync_copy(swbuf, selfw_hbm.at[pl.ds(r, 1)])

    @pl.loop(0, 79)
    def _(i):
        t = s + 16 * i

        @pl.when(t < EC // 2)
        def _():
            chunk = c * (EC // 2) + t
            pltpu.sync_copy(row_hbm.at[pl.ds(chunk, 1)], rowbuf)
            pltpu.sync_copy(col_hbm.at[pl.ds(chunk, 1)], colbuf)
            pltpu.sync_copy(ew_hbm.at[pl.ds(chunk, 1)], ewbuf)
            for j in range(8):
                sl = pl.ds(16 * j, 16)
                dr = plsc.load_gather(dinvbuf, [rowbuf[0, sl]])
                dc = plsc.load_gather(dinvbuf, [colbuf[0, sl]])
                nbuf[0, sl] = dr * ewbuf[0, sl] * dc
            pltpu.sync_copy(nbuf, norm_hbm.at[pl.ds(chunk, 1)])


@jax.jit
def _prep(row2d, col2d, ew2d):
    return pl.kernel(
        _prep_body,
        out_type=(jax.ShapeDtypeStruct((EC, CH), jnp.float32),
                  jax.ShapeDtypeStruct((NP // CH, CH), jnp.float32)),
        mesh=_mesh,
        scratch_types=[
            pltpu.VMEM_SHARED((NP,), jnp.float32),
            pltpu.VMEM((640,), jnp.float32),
            pltpu.VMEM((1, CH), jnp.int32),
            pltpu.VMEM((1, CH), jnp.float32),
            pltpu.VMEM((1, CH), jnp.int32),
            pltpu.VMEM((1, CH), jnp.float32),
            pltpu.VMEM((1, CH), jnp.float32),
            pltpu.VMEM((NP,), jnp.float32),
            pltpu.VMEM((NP,), jnp.float32),
        ],
        compiler_params=_sc_params,
    )(row2d, col2d, ew2d)


def _layer_body(xw_hbm, rn_hbm, col_hbm, selfw_hbm, b_hbm,
                h_hbm, acc_sp, rows0, rows1,
                rn0, rn1, rn2, col0, col1, col2,
                swbuf, bbuf,
                rnsem0, rnsem1, rnsem2, csem0, csem1, csem2,
                gsem0, gsem1, ssem0, ssem1):
    s = lax.axis_index("s")
    c = lax.axis_index("c")
    rows = (rows0, rows1)
    rnb = (rn0, rn1, rn2)
    colb = (col0, col1, col2)
    rnsem = (rnsem0, rnsem1, rnsem2)
    csem = (csem0, csem1, csem2)
    gsem = (gsem0, gsem1)
    ssem = (ssem0, ssem1)

    _zero_vmem2d(rows0, CH)
    for m in range(5):
        pltpu.sync_copy(rows0, acc_sp.at[pl.ds(s * 640 + m * 128, 128)])
    pltpu.sync_copy(b_hbm.at[pl.ds(c, 1)], bbuf)
    plsc.subcore_barrier()

    def t_of(i):
        return s + 16 * i

    def rn_issue(i, e3):
        @pl.when(t_of(i) < SCN)
        def _():
            pltpu.make_async_copy(
                rn_hbm.at[pl.ds(G * t_of(i), G)], rnb[e3], rnsem[e3]).start()

    def col_issue(i, e3):
        @pl.when(t_of(i) < SCN)
        def _():
            pltpu.make_async_copy(
                col_hbm.at[pl.ds(G * t_of(i), G)], colb[e3], csem[e3]).start()

    def rn_wait(i, e3):
        @pl.when(t_of(i) < SCN)
        def _():
            pltpu.make_async_copy(
                rn_hbm.at[pl.ds(G * t_of(i), G)], rnb[e3], rnsem[e3]).wait()

    def gather_issue(i, e3, p2):
        @pl.when(t_of(i) < SCN)
        def _():
            for g in range(G):
                pltpu.make_async_copy(
                    xw_hbm.at[c].at[rnb[e3].at[g, 0]],
                    rows[p2].at[pl.ds(CH * g, CH)], gsem[p2]).start()

    def gather_wait(i, p2):
        @pl.when(t_of(i) < SCN)
        def _():
            for g in range(G):
                pltpu.make_async_copy(
                    xw_hbm.at[c].at[rnb[0].at[g, 0]],
                    rows[p2].at[pl.ds(CH * g, CH)], gsem[p2]).wait()

    def scatter_issue(i, e3, p2):
        @pl.when(t_of(i) < SCN)
        def _():
            pltpu.make_async_copy(
                col_hbm.at[pl.ds(G * t_of(i), G)], colb[e3], csem[e3]).wait()
            for g in range(G):
                pltpu.async_copy(
                    rows[p2].at[pl.ds(CH * g, CH)],
                    acc_sp.at[colb[e3].at[g]], ssem[p2], add=True)

    def scatter_wait(i, p2):
        @pl.when((i >= 0) & (t_of(i) < SCN))
        def _():
            for g in range(G):
                pltpu.make_async_copy(
                    rows[p2].at[pl.ds(CH * g, CH)],
                    acc_sp.at[colb[0].at[g]], ssem[p2]).wait()

    def scale(e3, p2):
        @pl.loop(0, 8)
        def _(sub):
            nvec = plsc.bitcast(rnb[e3][0, 1, pl.ds(16 * sub, 16)],
                                jnp.float32)
            for l in range(16):
                ns = nvec[l]
                e = 16 * sub + l
                for j in range(8):
                    sl = pl.ds(16 * j, 16)
                    rows[p2][e, sl] = rows[p2][e, sl] * ns

    for ii in range(3):
        rn_issue(ii, ii)
    for ii in range(2):
        col_issue(ii, ii)
    rn_wait(0, 0)
    gather_issue(0, 0, 0)

    @pl.loop(0, 27)
    def _(blk):
        for u in range(6):
            i = blk * 6 + u
            p2 = u % 2
            e3 = u % 3
            gather_wait(i, p2)
            scatter_wait(i - 1, 1 - p2)
            col_issue(i + 2, (e3 + 2) % 3)
            rn_wait(i + 1, (e3 + 1) % 3)
            gather_issue(i + 1, (e3 + 1) % 3, 1 - p2)
            scale(e3, p2)
            scatter_issue(i, e3, p2)
            rn_issue(i + 3, e3)

    plsc.subcore_barrier()

    for m in range(5):
        r0 = s * 640 + m * 128
        pltpu.sync_copy(acc_sp.at[pl.ds(r0, 128)], rows0)
        pltpu.sync_copy(xw_hbm.at[c].at[pl.ds(r0, 128)], rows1)
        pltpu.sync_copy(selfw_hbm.at[pl.ds(s * 5 + m, 1)], swbuf)

        @pl.loop(0, CH // 16)
        def _(g):
            swv = swbuf[0, pl.ds(16 * g, 16)]
            for l in range(16):
                sw = swv[l]
                e = 16 * g + l
                for j in range(8):
                    sl = pl.ds(16 * j, 16)
                    v = rows0[e, sl] + sw * rows1[e, sl] + bbuf[0, sl]
                    rows0[e, sl] = jnp.maximum(v, 0.0)

        pltpu.sync_copy(rows0, h_hbm.at[c].at[pl.ds(r0, 128)])


@jax.jit
def _layer(xw, rn, col2d, selfw, b2d):
    return pl.kernel(
        _layer_body,
        out_type=jax.ShapeDtypeStruct((NCORE, NP, 128), jnp.float32),
        mesh=_mesh,
        scratch_types=[
            pltpu.VMEM_SHARED((NP, 128), jnp.float32),
            pltpu.VMEM((G * CH, 128), jnp.float32),
            pltpu.VMEM((G * CH, 128), jnp.float32),
            pltpu.VMEM((G, 2, CH), jnp.int32),
            pltpu.VMEM((G, 2, CH), jnp.int32),
            pltpu.VMEM((G, 2, CH), jnp.int32),
            pltpu.VMEM((G, CH), jnp.int32),
            pltpu.VMEM((G, CH), jnp.int32),
            pltpu.VMEM((G, CH), jnp.int32),
            pltpu.VMEM((1, CH), jnp.float32),
            pltpu.VMEM((1, 128), jnp.float32),
            pltpu.SemaphoreType.DMA,
            pltpu.SemaphoreType.DMA,
            pltpu.SemaphoreType.DMA,
            pltpu.SemaphoreType.DMA,
            pltpu.SemaphoreType.DMA,
            pltpu.SemaphoreType.DMA,
            pltpu.SemaphoreType.DMA,
            pltpu.SemaphoreType.DMA,
            pltpu.SemaphoreType.DMA,
            pltpu.SemaphoreType.DMA,
        ],
        compiler_params=_sc_params,
    )(xw, rn, col2d, selfw, b2d)


def _mm_split_body(a_ref, w_ref, o_ref):
    k = pl.program_id(2)

    @pl.when(k == 0)
    def _():
        o_ref[...] = jnp.zeros_like(o_ref)

    o_ref[0] += jnp.dot(a_ref[0], w_ref[0, 0],
                        preferred_element_type=jnp.float32)


@jax.jit
def _mm_split(parts, w4):
    kin = parts.shape[0]
    return pl.pallas_call(
        _mm_split_body,
        grid=(NB, NCORE, kin),
        in_specs=[
            pl.BlockSpec((1, BN, 128), lambda i, c, k: (k, i, 0)),
            pl.BlockSpec((1, 1, 128, 128), lambda i, c, k: (k, c, 0, 0)),
        ],
        out_specs=pl.BlockSpec((1, BN, 128), lambda i, c, k: (c, i, 0)),
        out_shape=jax.ShapeDtypeStruct((NCORE, NP, 128), jnp.float32),
    )(parts, w4)


def _mm_head_body(p_ref, w_ref, b_ref, o_ref):
    k = pl.program_id(1)

    @pl.when(k == 0)
    def _():
        o_ref[...] = jnp.broadcast_to(b_ref[...], o_ref.shape)

    o_ref[...] += jnp.dot(p_ref[0], w_ref[0],
                          preferred_element_type=jnp.float32)


@jax.jit
def _mm_head(parts, w, b):
    return pl.pallas_call(
        _mm_head_body,
        grid=(NB, 4),
        in_specs=[
            pl.BlockSpec((1, BN, 128), lambda i, k: (k, i, 0)),
            pl.BlockSpec((1, 128, 128), lambda i, k: (k, 0, 0)),
            pl.BlockSpec((1, 128), lambda i, k: (0, 0)),
        ],
        out_specs=pl.BlockSpec((BN, 128), lambda i, k: (i, 0)),
        out_shape=jax.ShapeDtypeStruct((NP, 128), jnp.float32),
    )(parts, w, b)


def kernel(x, edge_index, edge_weight, W1, b1, W2, b2, W3, b3, Wfc, bfc):
    row2d = edge_index[0].reshape(EC, CH)
    col2d = edge_index[1].reshape(EC, CH)
    ew2d = edge_weight.reshape(EC, CH)

    norm2d, selfw = _prep(row2d, col2d, ew2d)
    rn = jnp.stack(
        [row2d, jax.lax.bitcast_convert_type(norm2d, jnp.int32)], axis=1)

    xp = jnp.zeros((1, NP, FIN), jnp.float32).at[0, :N, :].set(x)
    xw1 = _mm_split(xp, W1.reshape(1, 128, 2, 128).transpose(0, 2, 1, 3))
    h1 = _layer(xw1, rn, col2d, selfw, b1.reshape(2, 128))

    xw2 = _mm_split(h1, W2.reshape(2, 128, 2, 128).transpose(0, 2, 1, 3))
    h2 = _layer(xw2, rn, col2d, selfw, b2.reshape(2, 128))

    xw3 = _mm_split(h2, W3.reshape(2, 128, 2, 128).transpose(0, 2, 1, 3))
    h3 = _layer(xw3, rn, col2d, selfw, b3.reshape(2, 128))

    parts = jnp.concatenate([h2, h3], axis=0)
    wfc = jnp.zeros((4, 128, 128), jnp.float32).at[:, :, :OUT].set(
        Wfc.reshape(4, 128, OUT))
    bfc_pad = jnp.zeros((1, 128), jnp.float32).at[0, :OUT].set(bfc)
    out_pad = _mm_head(parts, wfc, bfc_pad)
    return out_pad[:N, :OUT]


# speedup vs baseline: 14.0545x; 1.2272x over previous
import dataclasses
import functools

import jax
import jax.numpy as jnp
from jax import lax
from jax.experimental import pallas as pl
from jax.experimental.pallas import tpu as pltpu
from jax.experimental.pallas import tpu_sc as plsc

N = 10000
E = 320000
FIN = 128
H = 256
OUT = 2

NP = 10240
CH = 128
EC = E // CH
G = 1
SCN = EC // G
NSUB = 16
NCORE = 2
BN = 1024
NB = NP // BN

_mesh = plsc.VectorSubcoreMesh(core_axis_name="c", subcore_axis_name="s")

_sc_params = pltpu.CompilerParams()
if "needs_layout_passes" in pltpu.CompilerParams.__dataclass_fields__:
    _sc_params = dataclasses.replace(_sc_params, needs_layout_passes=False)


def _zero_vmem2d(buf, rows):
    z = jnp.zeros((16,), jnp.float32)

    @pl.loop(0, rows)
    def _(e):
        for j in range(8):
            buf[e, pl.ds(16 * j, 16)] = z


def _rsqrt16(d):
    i = plsc.bitcast(d, jnp.int32)
    i = 0x5F3759DF - lax.shift_right_arithmetic(i, 1)
    y = plsc.bitcast(i, jnp.float32)
    for _ in range(3):
        y = y * (1.5 - 0.5 * d * y * y)
    return y


def _prep_body(row_hbm, col_hbm, ew_hbm, norm_hbm, selfw_hbm,
               deg_sp, zbuf, col0, col1, ewb0, ewb1, row0, row1,
               nb0, nb1, swbuf, degbuf, dinvbuf,
               lsem0, lsem1, ssem0, ssem1, wsem0, wsem1):
    s = lax.axis_index("s")
    c = lax.axis_index("c")
    colb = (col0, col1)
    ewb = (ewb0, ewb1)
    rowb = (row0, row1)
    nb = (nb0, nb1)
    lsem = (lsem0, lsem1)
    ssem = (ssem0, ssem1)
    wsem = (wsem0, wsem1)

    @pl.loop(0, 40)
    def _(i):
        zbuf[pl.ds(16 * i, 16)] = jnp.zeros((16,), jnp.float32)

    pltpu.sync_copy(zbuf, deg_sp.at[pl.ds(s * 640, 640)])
    plsc.subcore_barrier()

    DB = 4
    DSC = EC // DB

    def d_t(i):
        return s + 16 * i

    def dload_issue(i, p):
        @pl.when(d_t(i) < DSC)
        def _():
            pltpu.make_async_copy(
                col_hbm.at[pl.ds(DB * d_t(i), DB)], colb[p], lsem[p]).start()
            pltpu.make_async_copy(
                ew_hbm.at[pl.ds(DB * d_t(i), DB)], ewb[p], lsem[p]).start()

    def dload_wait(i, p):
        @pl.when(d_t(i) < DSC)
        def _():
            pltpu.make_async_copy(
                col_hbm.at[pl.ds(DB * d_t(i), DB)], colb[p], lsem[p]).wait()
            pltpu.make_async_copy(
                ew_hbm.at[pl.ds(DB * d_t(i), DB)], ewb[p], lsem[p]).wait()

    def dscat_issue(i, p):
        @pl.when(d_t(i) < DSC)
        def _():
            for g in range(DB):
                pltpu.async_copy(ewb[p].at[g], deg_sp.at[colb[p].at[g]],
                                 ssem[p], add=True)

    def dscat_wait(i, p):
        @pl.when((i >= 0) & (d_t(i) < DSC))
        def _():
            for g in range(DB):
                pltpu.make_async_copy(ewb[p].at[g], deg_sp.at[colb[0].at[g]],
                                      ssem[p]).wait()

    dload_issue(0, 0)

    @pl.loop(0, 21)
    def _(blk):
        for u in range(2):
            i = blk * 2 + u
            p = u % 2
            dscat_wait(i - 1, 1 - p)
            dload_issue(i + 1, 1 - p)
            dload_wait(i, p)
            dscat_issue(i, p)

    plsc.subcore_barrier()

    pltpu.sync_copy(deg_sp, degbuf)

    @pl.loop(0, NP // 16)
    def _(i):
        d = degbuf[pl.ds(16 * i, 16)] + 1.0
        dinvbuf[pl.ds(16 * i, 16)] = _rsqrt16(d)

    @pl.when(c == 0)
    def _():
        for m in range(5):
            r = s * 5 + m
            for j in range(8):
                v = dinvbuf[pl.ds(r * 128 + 16 * j, 16)]
                swbuf[0, pl.ds(16 * j, 16)] = v * v
            pltpu.sync_copy(swbuf, selfw_hbm.at[pl.ds(r, 1)])

    NBC = 2
    NSC = (EC // 2) // NBC

    def n_base(i):
        return c * (EC // 2) + NBC * (s + 16 * i)

    def nload_issue(i, p):
        @pl.when(s + 16 * i < NSC)
        def _():
            b = n_base(i)
            pltpu.make_async_copy(
                row_hbm.at[pl.ds(b, NBC)], rowb[p], lsem[p]).start()
            pltpu.make_async_copy(
                col_hbm.at[pl.ds(b, NBC)], colb[p].at[pl.ds(0, NBC)],
                lsem[p]).start()
            pltpu.make_async_copy(
                ew_hbm.at[pl.ds(b, NBC)], ewb[p].at[pl.ds(0, NBC)],
                lsem[p]).start()

    def nload_wait(i, p):
        @pl.when(s + 16 * i < NSC)
        def _():
            b = n_base(i)
            pltpu.make_async_copy(
                row_hbm.at[pl.ds(b, NBC)], rowb[p], lsem[p]).wait()
            pltpu.make_async_copy(
                col_hbm.at[pl.ds(b, NBC)], colb[p].at[pl.ds(0, NBC)],
                lsem[p]).wait()
            pltpu.make_async_copy(
                ew_hbm.at[pl.ds(b, NBC)], ewb[p].at[pl.ds(0, NBC)],
                lsem[p]).wait()

    def nwrite_issue(i, p):
        @pl.when(s + 16 * i < NSC)
        def _():
            pltpu.async_copy(nb[p], norm_hbm.at[pl.ds(n_base(i), NBC)],
                             wsem[p])

    def nwrite_wait(i, p):
        @pl.when((i >= 0) & (s + 16 * i < NSC))
        def _():
            pltpu.make_async_copy(nb[p], norm_hbm.at[pl.ds(0, NBC)],
                                  wsem[p]).wait()

    def ncompute(i, p):
        @pl.when(s + 16 * i < NSC)
        def _():
            for g in range(NBC):
                for j in range(8):
                    sl = pl.ds(16 * j, 16)
                    dr = plsc.load_gather(dinvbuf, [rowb[p][g, sl]])
                    dc = plsc.load_gather(dinvbuf, [colb[p][g, sl]])
                    nb[p][g, sl] = dr * ewb[p][g, sl] * dc

    nload_issue(0, 0)

    @pl.loop(0, 21)
    def _(blk):
        for u in range(2):
            i = blk * 2 + u
            p = u % 2
            nwrite_wait(i - 2, p)
            nload_wait(i, p)
            nload_issue(i + 1, 1 - p)
            ncompute(i, p)
            nwrite_issue(i, p)


@jax.jit
def _prep(row2d, col2d, ew2d):
    return pl.kernel(
        _prep_body,
        out_type=(jax.ShapeDtypeStruct((EC, CH), jnp.float32),
                  jax.ShapeDtypeStruct((NP // CH, CH), jnp.float32)),
        mesh=_mesh,
        scratch_types=[
            pltpu.VMEM_SHARED((NP,), jnp.float32),
            pltpu.VMEM((640,), jnp.float32),
            pltpu.VMEM((4, CH), jnp.int32),
            pltpu.VMEM((4, CH), jnp.int32),
            pltpu.VMEM((4, CH), jnp.float32),
            pltpu.VMEM((4, CH), jnp.float32),
            pltpu.VMEM((2, CH), jnp.int32),
            pltpu.VMEM((2, CH), jnp.int32),
            pltpu.VMEM((2, CH), jnp.float32),
            pltpu.VMEM((2, CH), jnp.float32),
            pltpu.VMEM((1, CH), jnp.float32),
            pltpu.VMEM((NP,), jnp.float32),
            pltpu.VMEM((NP,), jnp.float32),
            pltpu.SemaphoreType.DMA,
            pltpu.SemaphoreType.DMA,
            pltpu.SemaphoreType.DMA,
            pltpu.SemaphoreType.DMA,
            pltpu.SemaphoreType.DMA,
            pltpu.SemaphoreType.DMA,
        ],
        compiler_params=_sc_params,
    )(row2d, col2d, ew2d)


def _layer_body(xw_hbm, rn_hbm, col_hbm, selfw_hbm, b_hbm,
                h_hbm, acc_sp, rows0, rows1,
                rn0, rn1, rn2, col0, col1, col2,
                swbuf, bbuf,
                rnsem0, rnsem1, rnsem2, csem0, csem1, csem2,
                gsem0, gsem1, ssem0, ssem1):
    s = lax.axis_index("s")
    c = lax.axis_index("c")
    rows = (rows0, rows1)
    rnb = (rn0, rn1, rn2)
    colb = (col0, col1, col2)
    rnsem = (rnsem0, rnsem1, rnsem2)
    csem = (csem0, csem1, csem2)
    gsem = (gsem0, gsem1)
    ssem = (ssem0, ssem1)

    _zero_vmem2d(rows0, CH)
    for m in range(5):
        pltpu.sync_copy(rows0, acc_sp.at[pl.ds(s * 640 + m * 128, 128)])
    pltpu.sync_copy(b_hbm.at[pl.ds(c, 1)], bbuf)
    plsc.subcore_barrier()

    def t_of(i):
        return s + 16 * i

    def rn_issue(i, e3):
        @pl.when(t_of(i) < SCN)
        def _():
            pltpu.make_async_copy(
                rn_hbm.at[pl.ds(G * t_of(i), G)], rnb[e3], rnsem[e3]).start()

    def col_issue(i, e3):
        @pl.when(t_of(i) < SCN)
        def _():
            pltpu.make_async_copy(
                col_hbm.at[pl.ds(G * t_of(i), G)], colb[e3], csem[e3]).start()

    def rn_wait(i, e3):
        @pl.when(t_of(i) < SCN)
        def _():
            pltpu.make_async_copy(
                rn_hbm.at[pl.ds(G * t_of(i), G)], rnb[e3], rnsem[e3]).wait()

    def gather_issue(i, e3, p2):
        @pl.when(t_of(i) < SCN)
        def _():
            for g in range(G):
                pltpu.make_async_copy(
                    xw_hbm.at[c].at[rnb[e3].at[g, 0]],
                    rows[p2].at[pl.ds(CH * g, CH)], gsem[p2]).start()

    def gather_wait(i, p2):
        @pl.when(t_of(i) < SCN)
        def _():
            for g in range(G):
                pltpu.make_async_copy(
                    xw_hbm.at[c].at[rnb[0].at[g, 0]],
                    rows[p2].at[pl.ds(CH * g, CH)], gsem[p2]).wait()

    def scatter_issue(i, e3, p2):
        @pl.when(t_of(i) < SCN)
        def _():
            pltpu.make_async_copy(
                col_hbm.at[pl.ds(G * t_of(i), G)], colb[e3], csem[e3]).wait()
            for g in range(G):
                pltpu.async_copy(
                    rows[p2].at[pl.ds(CH * g, CH)],
                    acc_sp.at[colb[e3].at[g]], ssem[p2], add=True)

    def scatter_wait(i, p2):
        @pl.when((i >= 0) & (t_of(i) < SCN))
        def _():
            for g in range(G):
                pltpu.make_async_copy(
                    rows[p2].at[pl.ds(CH * g, CH)],
                    acc_sp.at[colb[0].at[g]], ssem[p2]).wait()

    def scale(e3, p2):
        @pl.loop(0, 8)
        def _(sub):
            nvec = plsc.bitcast(rnb[e3][0, 1, pl.ds(16 * sub, 16)],
                                jnp.float32)
            for l in range(16):
                ns = nvec[l]
                e = 16 * sub + l
                for j in range(8):
                    sl = pl.ds(16 * j, 16)
                    rows[p2][e, sl] = rows[p2][e, sl] * ns

    for ii in range(3):
        rn_issue(ii, ii)
    for ii in range(2):
        col_issue(ii, ii)
    rn_wait(0, 0)
    gather_issue(0, 0, 0)

    @pl.loop(0, 27)
    def _(blk):
        for u in range(6):
            i = blk * 6 + u
            p2 = u % 2
            e3 = u % 3
            gather_wait(i, p2)
            scatter_wait(i - 1, 1 - p2)
            col_issue(i + 2, (e3 + 2) % 3)
            rn_wait(i + 1, (e3 + 1) % 3)
            gather_issue(i + 1, (e3 + 1) % 3, 1 - p2)
            scale(e3, p2)
            scatter_issue(i, e3, p2)
            rn_issue(i + 3, e3)

    plsc.subcore_barrier()

    for m in range(5):
        r0 = s * 640 + m * 128
        pltpu.sync_copy(acc_sp.at[pl.ds(r0, 128)], rows0)
        pltpu.sync_copy(xw_hbm.at[c].at[pl.ds(r0, 128)], rows1)
        pltpu.sync_copy(selfw_hbm.at[pl.ds(s * 5 + m, 1)], swbuf)

        @pl.loop(0, CH // 16)
        def _(g):
            swv = swbuf[0, pl.ds(16 * g, 16)]
            for l in range(16):
                sw = swv[l]
                e = 16 * g + l
                for j in range(8):
                    sl = pl.ds(16 * j, 16)
                    v = rows0[e, sl] + sw * rows1[e, sl] + bbuf[0, sl]
                    rows0[e, sl] = jnp.maximum(v, 0.0)

        pltpu.sync_copy(rows0, h_hbm.at[c].at[pl.ds(r0, 128)])


@jax.jit
def _layer(xw, rn, col2d, selfw, b2d):
    return pl.kernel(
        _layer_body,
        out_type=jax.ShapeDtypeStruct((NCORE, NP, 128), jnp.float32),
        mesh=_mesh,
        scratch_types=[
            pltpu.VMEM_SHARED((NP, 128), jnp.float32),
            pltpu.VMEM((G * CH, 128), jnp.float32),
            pltpu.VMEM((G * CH, 128), jnp.float32),
            pltpu.VMEM((G, 2, CH), jnp.int32),
            pltpu.VMEM((G, 2, CH), jnp.int32),
            pltpu.VMEM((G, 2, CH), jnp.int32),
            pltpu.VMEM((G, CH), jnp.int32),
            pltpu.VMEM((G, CH), jnp.int32),
            pltpu.VMEM((G, CH), jnp.int32),
            pltpu.VMEM((1, CH), jnp.float32),
            pltpu.VMEM((1, 128), jnp.float32),
            pltpu.SemaphoreType.DMA,
            pltpu.SemaphoreType.DMA,
            pltpu.SemaphoreType.DMA,
            pltpu.SemaphoreType.DMA,
            pltpu.SemaphoreType.DMA,
            pltpu.SemaphoreType.DMA,
            pltpu.SemaphoreType.DMA,
            pltpu.SemaphoreType.DMA,
            pltpu.SemaphoreType.DMA,
            pltpu.SemaphoreType.DMA,
        ],
        compiler_params=_sc_params,
    )(xw, rn, col2d, selfw, b2d)


def _mm_split_body(a_ref, w_ref, o_ref):
    k = pl.program_id(2)

    @pl.when(k == 0)
    def _():
        o_ref[...] = jnp.zeros_like(o_ref)

    o_ref[0] += jnp.dot(a_ref[0], w_ref[0, 0],
                        preferred_element_type=jnp.float32)


@jax.jit
def _mm_split(parts, w4):
    kin = parts.shape[0]
    return pl.pallas_call(
        _mm_split_body,
        grid=(NB, NCORE, kin),
        in_specs=[
            pl.BlockSpec((1, BN, 128), lambda i, c, k: (k, i, 0)),
            pl.BlockSpec((1, 1, 128, 128), lambda i, c, k: (k, c, 0, 0)),
        ],
        out_specs=pl.BlockSpec((1, BN, 128), lambda i, c, k: (c, i, 0)),
        out_shape=jax.ShapeDtypeStruct((NCORE, NP, 128), jnp.float32),
    )(parts, w4)


def _mm_head_body(p_ref, w_ref, b_ref, o_ref):
    k = pl.program_id(1)

    @pl.when(k == 0)
    def _():
        o_ref[...] = jnp.broadcast_to(b_ref[...], o_ref.shape)

    o_ref[...] += jnp.dot(p_ref[0], w_ref[0],
                          preferred_element_type=jnp.float32)


@jax.jit
def _mm_head(parts, w, b):
    return pl.pallas_call(
        _mm_head_body,
        grid=(NB, 4),
        in_specs=[
            pl.BlockSpec((1, BN, 128), lambda i, k: (k, i, 0)),
            pl.BlockSpec((1, 128, 128), lambda i, k: (k, 0, 0)),
            pl.BlockSpec((1, 128), lambda i, k: (0, 0)),
        ],
        out_specs=pl.BlockSpec((BN, 128), lambda i, k: (i, 0)),
        out_shape=jax.ShapeDtypeStruct((NP, 128), jnp.float32),
    )(parts, w, b)


def kernel(x, edge_index, edge_weight, W1, b1, W2, b2, W3, b3, Wfc, bfc):
    row2d = edge_index[0].reshape(EC, CH)
    col2d = edge_index[1].reshape(EC, CH)
    ew2d = edge_weight.reshape(EC, CH)

    norm2d, selfw = _prep(row2d, col2d, ew2d)
    rn = jnp.stack(
        [row2d, jax.lax.bitcast_convert_type(norm2d, jnp.int32)], axis=1)

    xp = jnp.zeros((1, NP, FIN), jnp.float32).at[0, :N, :].set(x)
    xw1 = _mm_split(xp, W1.reshape(1, 128, 2, 128).transpose(0, 2, 1, 3))
    h1 = _layer(xw1, rn, col2d, selfw, b1.reshape(2, 128))

    xw2 = _mm_split(h1, W2.reshape(2, 128, 2, 128).transpose(0, 2, 1, 3))
    h2 = _layer(xw2, rn, col2d, selfw, b2.reshape(2, 128))

    xw3 = _mm_split(h2, W3.reshape(2, 128, 2, 128).transpose(0, 2, 1, 3))
    h3 = _layer(xw3, rn, col2d, selfw, b3.reshape(2, 128))

    parts = jnp.concatenate([h2, h3], axis=0)
    wfc = jnp.zeros((4, 128, 128), jnp.float32).at[:, :, :OUT].set(
        Wfc.reshape(4, 128, OUT))
    bfc_pad = jnp.zeros((1, 128), jnp.float32).at[0, :OUT].set(bfc)
    out_pad = _mm_head(parts, wfc, bfc_pad)
    return out_pad[:N, :OUT]
